# baseline (speedup 1.0000x reference)
import functools

import jax
import jax.numpy as jnp
import numpy as np
from jax.experimental import pallas as pl
from jax.experimental.pallas import tpu as pltpu

_C = 192
_NMIX = _C // 2

_mix_idx = np.asarray(jax.random.permutation(jax.random.key(42), _C))[:_NMIX]
_swap_mask = np.zeros(_C, dtype=bool)
_swap_mask[_mix_idx] = True
_ORDER = np.concatenate(
    [np.where(~_swap_mask)[0], np.where(_swap_mask)[0]]
).astype(np.int32)
_NKEEP = int((~_swap_mask).sum())


def _body(order_ref, f1_ref, f2_ref, o_ref):
    i = pl.program_id(0)

    @pl.when(i < _NKEEP)
    def _():
        o_ref[:, 0] = f1_ref[...]
        o_ref[:, 1] = f2_ref[...]

    @pl.when(i >= _NKEEP)
    def _():
        o_ref[:, 0] = f2_ref[...]
        o_ref[:, 1] = f1_ref[...]


@functools.partial(jax.jit)
def kernel(f1, f2):
    B, C, H, W = f1.shape
    HW = H * W
    LANES = 128
    ROWS = HW // LANES
    a = f1.reshape(B, C, ROWS, LANES)
    b = f2.reshape(B, C, ROWS, LANES)

    grid_spec = pltpu.PrefetchScalarGridSpec(
        num_scalar_prefetch=1,
        grid=(C,),
        in_specs=[
            pl.BlockSpec((B, 1, ROWS, LANES), lambda i, order: (0, order[i], 0, 0)),
            pl.BlockSpec((B, 1, ROWS, LANES), lambda i, order: (0, order[i], 0, 0)),
        ],
        out_specs=pl.BlockSpec(
            (B, 2, 1, ROWS, LANES), lambda i, order: (0, 0, order[i], 0, 0)
        ),
    )
    out = pl.pallas_call(
        _body,
        grid_spec=grid_spec,
        out_shape=jax.ShapeDtypeStruct((B, 2, C, ROWS, LANES), f1.dtype),
        compiler_params=pltpu.CompilerParams(
            dimension_semantics=("arbitrary",),
        ),
    )(jnp.asarray(_ORDER), a, b)
    return out.reshape(B, 2 * C, H, W)
